# Initial kernel scaffold; baseline (speedup 1.0000x reference)
import jax
import jax.numpy as jnp
from jax.experimental import pallas as pl


def kernel(h_query, memory_embeds, true_values, R):
    raise NotImplementedError("write your pallas kernel here")


# speedup vs baseline: 3.0458x; 3.0458x over previous
import functools

import jax
import jax.numpy as jnp
from jax.experimental import pallas as pl
from jax.experimental.pallas import tpu as pltpu

Q = 1024
D2 = 128
NBITS = 32
TOPK = 32
KB = 1024


def _prefix_incl(x):
    n = x.shape[1]
    s = 1
    while s < n:
        shifted = jnp.concatenate(
            [jnp.zeros((x.shape[0], s), x.dtype), x[:, : n - s]], axis=1)
        x = x + shifted
        s *= 2
    return x


def _body(nblk, k_real, q_ref, r_ref, m_ref, tv_ref, y_ref,
          qc_ref, hist_ref, vt_ref, rr_ref, runcnt_ref, acc_ref):
    p = pl.program_id(0)
    i = pl.program_id(1)

    @pl.when((p == 0) & (i == 0))
    def _init():
        proj_q = jax.lax.dot_general(
            q_ref[...].astype(jnp.bfloat16), r_ref[...].astype(jnp.bfloat16),
            (((1,), (0,)), ((), ())), preferred_element_type=jnp.float32)
        qc_ref[...] = jnp.where(proj_q > 0, 1.0, -1.0)
        hist_ref[...] = jnp.zeros_like(hist_ref)

    proj_m = jax.lax.dot_general(
        m_ref[...].astype(jnp.bfloat16), r_ref[...].astype(jnp.bfloat16),
        (((1,), (0,)), ((), ())), preferred_element_type=jnp.float32)
    mc = jnp.where(proj_m > 0, 1.0, -1.0)
    sim = jax.lax.dot_general(
        qc_ref[...].astype(jnp.bfloat16), mc.astype(jnp.bfloat16),
        (((1,), (1,)), ((), ())), preferred_element_type=jnp.float32)
    col = jax.lax.broadcasted_iota(jnp.int32, sim.shape, 1) + i * KB
    sim = jnp.where(col < k_real, sim, -100.0)

    @pl.when(p == 0)
    def _phase0():
        cols = []
        for b in range(NBITS + 1):
            v = float(NBITS - 2 * b)
            cols.append(jnp.sum(jnp.where(sim == v, 1.0, 0.0), axis=1,
                                keepdims=True))
        hist_ref[...] += jnp.concatenate(cols, axis=1)

    @pl.when((p == 0) & (i == nblk - 1))
    def _threshold():
        hist = hist_ref[...]
        nb = NBITS + 1
        row = jax.lax.broadcasted_iota(jnp.int32, (nb, nb), 0)
        colm = jax.lax.broadcasted_iota(jnp.int32, (nb, nb), 1)
        tril = jnp.where(row <= colm, 1.0, 0.0)
        cum = jax.lax.dot_general(
            hist, tril, (((1,), (0,)), ((), ())),
            preferred_element_type=jnp.float32)
        ge = jnp.where(cum >= float(TOPK), 1.0, 0.0)
        t = float(nb) - jnp.sum(ge, axis=1, keepdims=True)
        binidx = jax.lax.broadcasted_iota(jnp.int32, hist.shape, 1).astype(
            jnp.float32)
        sel_t = jnp.where(binidx == t, 1.0, 0.0)
        cum_excl_t = jnp.sum(sel_t * (cum - hist), axis=1, keepdims=True)
        rr_ref[...] = float(TOPK) - cum_excl_t
        vt_ref[...] = float(NBITS) - 2.0 * t
        runcnt_ref[...] = jnp.zeros_like(runcnt_ref)
        acc_ref[...] = jnp.zeros_like(acc_ref)

    @pl.when(p == 1)
    def _phase1():
        tv = tv_ref[...]
        vt = vt_ref[...]
        acc_gt = jnp.sum(jnp.where(sim > vt, tv, 0.0), axis=1, keepdims=True)
        eqm = jnp.where(sim == vt, 1.0, 0.0)
        pre = _prefix_incl(eqm)
        sel = eqm * jnp.where(pre + runcnt_ref[...] <= rr_ref[...], 1.0, 0.0)
        acc_ref[...] += acc_gt + jnp.sum(sel * tv, axis=1, keepdims=True)
        runcnt_ref[...] += jnp.sum(eqm, axis=1, keepdims=True)

    @pl.when((p == 1) & (i == nblk - 1))
    def _finalize():
        y_ref[...] = acc_ref[...] * (1.0 / float(TOPK))


@jax.jit
def kernel(h_query, memory_embeds, true_values, R):
    q, d2 = h_query.shape
    k_real = memory_embeds.shape[0]
    nblk = (k_real + KB - 1) // KB
    k_pad = nblk * KB
    if k_pad != k_real:
        memory_embeds = jnp.pad(memory_embeds, ((0, k_pad - k_real), (0, 0)))
        true_values = jnp.pad(true_values, (0, k_pad - k_real))
    tv2 = true_values.reshape(1, k_pad)

    y = pl.pallas_call(
        functools.partial(_body, nblk, k_real),
        grid=(2, nblk),
        in_specs=[
            pl.BlockSpec((q, d2), lambda p, i: (0, 0)),
            pl.BlockSpec((d2, NBITS), lambda p, i: (0, 0)),
            pl.BlockSpec((KB, d2), lambda p, i: (i, 0)),
            pl.BlockSpec((1, KB), lambda p, i: (0, i)),
        ],
        out_specs=pl.BlockSpec((q, 1), lambda p, i: (0, 0)),
        out_shape=jax.ShapeDtypeStruct((q, 1), jnp.float32),
        scratch_shapes=[
            pltpu.VMEM((q, NBITS), jnp.float32),
            pltpu.VMEM((q, NBITS + 1), jnp.float32),
            pltpu.VMEM((q, 1), jnp.float32),
            pltpu.VMEM((q, 1), jnp.float32),
            pltpu.VMEM((q, 1), jnp.float32),
            pltpu.VMEM((q, 1), jnp.float32),
        ],
        compiler_params=pltpu.CompilerParams(
            dimension_semantics=("arbitrary", "arbitrary")),
    )(h_query, R, memory_embeds, tv2)
    return y[:, 0]


# speedup vs baseline: 4.5257x; 1.4859x over previous
import functools

import jax
import jax.numpy as jnp
from jax.experimental import pallas as pl
from jax.experimental.pallas import tpu as pltpu

NBITS = 32
TOPK = 32
KB = 1024
GW = 6


def _prefix_incl(x):
    n = x.shape[1]
    s = 1
    while s < n:
        shifted = jnp.concatenate(
            [jnp.zeros((x.shape[0], s), x.dtype), x[:, : n - s]], axis=1)
        x = x + shifted
        s *= 2
    return x


def _body(nblk, k_real, q_ref, r_ref, m_ref, tv_ref, y_ref,
          qc_ref, mc_ref, cumA_ref, vB_ref, cumB_ref, jc_ref, cumbef_ref,
          vt_ref, rr_ref, runcnt_ref, acc_ref):
    p = pl.program_id(0)
    i = pl.program_id(1)
    qshape = qc_ref.shape[0]

    @pl.when((p == 0) & (i == 0))
    def _init():
        proj_q = jax.lax.dot_general(
            q_ref[...].astype(jnp.bfloat16), r_ref[...].astype(jnp.bfloat16),
            (((1,), (0,)), ((), ())), preferred_element_type=jnp.float32)
        qc_ref[...] = jnp.where(proj_q > 0, 1.0, -1.0).astype(jnp.bfloat16)
        cumA_ref[...] = jnp.zeros_like(cumA_ref)

    @pl.when(p == 0)
    def _codes():
        proj_m = jax.lax.dot_general(
            m_ref[...].astype(jnp.bfloat16), r_ref[...].astype(jnp.bfloat16),
            (((1,), (0,)), ((), ())), preferred_element_type=jnp.float32)
        mc_ref[pl.ds(i * KB, KB), :] = jnp.where(
            proj_m > 0, 1.0, -1.0).astype(jnp.bfloat16)

    mc = mc_ref[pl.ds(i * KB, KB), :]
    sim = jax.lax.dot_general(
        qc_ref[...], mc, (((1,), (1,)), ((), ())),
        preferred_element_type=jnp.float32)
    col = jax.lax.broadcasted_iota(jnp.int32, (1, KB), 1) + i * KB
    sim = jnp.where(col < k_real, sim, -100.0)

    @pl.when(p == 0)
    def _coarse():
        cols = []
        for j in range(5):
            v = float(NBITS - 2 * (GW * j + GW - 1))
            cols.append(jnp.sum(jnp.where(sim >= v, 1.0, 0.0), axis=1,
                                keepdims=True))
        cumA_ref[...] += jnp.concatenate(cols, axis=1)

    @pl.when((p == 0) & (i == nblk - 1))
    def _coarse_combine():
        cumA = cumA_ref[...]
        ge = jnp.where(cumA >= float(TOPK), 1.0, 0.0)
        jc = 5.0 - jnp.sum(ge, axis=1, keepdims=True)
        binj = jax.lax.broadcasted_iota(jnp.int32, cumA.shape, 1).astype(
            jnp.float32)
        cumbef_ref[...] = jnp.sum(
            jnp.where(binj == jc - 1.0, cumA, 0.0), axis=1, keepdims=True)
        cum_top = jnp.sum(
            jnp.where(binj == jc, cumA, 0.0), axis=1, keepdims=True)
        cum_top = jnp.where(jc == 5.0, float(k_real), cum_top)
        jc_ref[...] = jc
        cumB_ref[...] = jnp.concatenate(
            [jnp.zeros((qshape, 5), jnp.float32), cum_top], axis=1)
        vcols = []
        for d in range(5):
            b_d = jnp.minimum(float(GW) * jc + float(d), float(NBITS))
            vcols.append(float(NBITS) - 2.0 * b_d)
        vB_ref[...] = jnp.concatenate(vcols, axis=1)

    @pl.when(p == 1)
    def _fine():
        vB = vB_ref[...]
        cols = []
        for d in range(5):
            cols.append(jnp.sum(
                jnp.where(sim >= vB[:, d:d + 1], 1.0, 0.0), axis=1,
                keepdims=True))
        cumB_ref[:, :5] += jnp.concatenate(cols, axis=1)

    @pl.when((p == 1) & (i == nblk - 1))
    def _fine_combine():
        cumB = cumB_ref[...]
        ge = jnp.where(cumB >= float(TOPK), 1.0, 0.0)
        dstar = 6.0 - jnp.sum(ge, axis=1, keepdims=True)
        t = jnp.minimum(float(GW) * jc_ref[...] + dstar, float(NBITS))
        bind = jax.lax.broadcasted_iota(jnp.int32, cumB.shape, 1).astype(
            jnp.float32)
        prevcum = jnp.sum(
            jnp.where(bind == dstar - 1.0, cumB, 0.0), axis=1, keepdims=True)
        prevcum = prevcum + jnp.where(dstar == 0.0, cumbef_ref[...], 0.0)
        rr_ref[...] = float(TOPK) - prevcum
        vt_ref[...] = float(NBITS) - 2.0 * t
        runcnt_ref[...] = jnp.zeros_like(runcnt_ref)
        acc_ref[...] = jnp.zeros_like(acc_ref)

    @pl.when(p == 2)
    def _sums():
        tv = tv_ref[...]
        vt = vt_ref[...]
        acc_gt = jnp.sum(jnp.where(sim > vt, tv, 0.0), axis=1, keepdims=True)
        eqm = jnp.where(sim == vt, 1.0, 0.0)
        pre = _prefix_incl(eqm)
        sel = eqm * jnp.where(pre + runcnt_ref[...] <= rr_ref[...], 1.0, 0.0)
        acc_ref[...] += acc_gt + jnp.sum(sel * tv, axis=1, keepdims=True)
        runcnt_ref[...] += jnp.sum(eqm, axis=1, keepdims=True)

    @pl.when((p == 2) & (i == nblk - 1))
    def _finalize():
        y_ref[...] = acc_ref[...] * (1.0 / float(TOPK))


@jax.jit
def kernel(h_query, memory_embeds, true_values, R):
    q, d2 = h_query.shape
    k_real = memory_embeds.shape[0]
    nblk = (k_real + KB - 1) // KB
    k_pad = nblk * KB
    if k_pad != k_real:
        memory_embeds = jnp.pad(memory_embeds, ((0, k_pad - k_real), (0, 0)))
        true_values = jnp.pad(true_values, (0, k_pad - k_real))
    tv2 = true_values.reshape(1, k_pad)

    y = pl.pallas_call(
        functools.partial(_body, nblk, k_real),
        grid=(3, nblk),
        in_specs=[
            pl.BlockSpec((q, d2), lambda p, i: (0, 0)),
            pl.BlockSpec((d2, NBITS), lambda p, i: (0, 0)),
            pl.BlockSpec((KB, d2), lambda p, i: (i, 0)),
            pl.BlockSpec((1, KB), lambda p, i: (0, i)),
        ],
        out_specs=pl.BlockSpec((q, 1), lambda p, i: (0, 0)),
        out_shape=jax.ShapeDtypeStruct((q, 1), jnp.float32),
        scratch_shapes=[
            pltpu.VMEM((q, NBITS), jnp.bfloat16),
            pltpu.VMEM((k_pad, NBITS), jnp.bfloat16),
            pltpu.VMEM((q, 5), jnp.float32),
            pltpu.VMEM((q, 5), jnp.float32),
            pltpu.VMEM((q, 6), jnp.float32),
            pltpu.VMEM((q, 1), jnp.float32),
            pltpu.VMEM((q, 1), jnp.float32),
            pltpu.VMEM((q, 1), jnp.float32),
            pltpu.VMEM((q, 1), jnp.float32),
            pltpu.VMEM((q, 1), jnp.float32),
            pltpu.VMEM((q, 1), jnp.float32),
        ],
        compiler_params=pltpu.CompilerParams(
            dimension_semantics=("arbitrary", "arbitrary")),
    )(h_query, R, memory_embeds, tv2)
    return y[:, 0]
